# baseline (speedup 1.0000x reference)
import functools

import jax
import jax.numpy as jnp
from jax import lax
from jax.experimental import pallas as pl
from jax.experimental.pallas import tpu as pltpu
from jax.experimental.pallas import tpu_sc as plsc

_LANES = 16
_N_WORKERS = 32
_CHUNK_COLS = 128
_CHUNKS = 4
_PAT_TABLE = 1048
_SHIFT = 20


def _lookup_rows(buf, tb, r_lo, r_hi):
    @plsc.parallel_loop(r_lo, r_hi, unroll=4)
    def body(r):
        for s in range(_CHUNK_COLS // _LANES):
            sl = pl.ds(s * _LANES, _LANES)
            idx = lax.shift_right_logical(
                plsc.bitcast(buf[r, sl], jnp.int32), _SHIFT)
            buf[r, sl] = plsc.load_gather(tb, [idx])


def _run(xt, tbl0, tbl1, n_cat0, n_cat1, n_vals0):
    n_params, batch = xt.shape
    stripe = batch // _N_WORKERS
    mesh = plsc.VectorSubcoreMesh(core_axis_name="c", subcore_axis_name="s")

    @functools.partial(
        pl.kernel,
        mesh=mesh,
        out_type=jax.ShapeDtypeStruct(xt.shape, jnp.float32),
        scratch_types=[
            pltpu.VMEM((n_params, _CHUNK_COLS), jnp.float32),
            pltpu.VMEM((n_params, _CHUNK_COLS), jnp.float32),
            pltpu.VMEM((n_params, _CHUNK_COLS), jnp.float32),
            pltpu.VMEM((n_params, _CHUNK_COLS), jnp.float32),
            pltpu.VMEM((8,), jnp.float32),
            pltpu.VMEM((_LANES,), jnp.float32),
            pltpu.VMEM((_PAT_TABLE,), jnp.float32),
            pltpu.VMEM((_PAT_TABLE,), jnp.float32),
            pltpu.SemaphoreType.DMA,
            pltpu.SemaphoreType.DMA,
            pltpu.SemaphoreType.DMA,
            pltpu.SemaphoreType.DMA,
            pltpu.SemaphoreType.DMA,
            pltpu.SemaphoreType.DMA,
            pltpu.SemaphoreType.DMA,
            pltpu.SemaphoreType.DMA,
        ],
        compiler_params=pltpu.CompilerParams(needs_layout_passes=False),
    )
    def body(xt_hbm, t0_hbm, t1_hbm, out_hbm, b0, b1, b2, b3, tv0, tv1,
             tb0, tb1, si0, si1, si2, si3, so0, so1, so2, so3):
        wid = lax.axis_index("s") * 2 + lax.axis_index("c")
        base = wid * stripe
        bufs = (b0, b1, b2, b3)
        sins = (si0, si1, si2, si3)
        souts = (so0, so1, so2, so3)

        pltpu.sync_copy(t0_hbm, tv0)
        pltpu.sync_copy(t1_hbm, tv1)

        ins = [
            pltpu.async_copy(
                xt_hbm.at[:, pl.ds(base + k * _CHUNK_COLS, _CHUNK_COLS)],
                bufs[k], sins[k])
            for k in range(_CHUNKS)
        ]

        lanes = lax.iota(jnp.int32, _LANES)
        pats = lax.shift_right_logical(
            plsc.bitcast(lanes.astype(jnp.float32), jnp.int32), _SHIFT)
        vals0 = plsc.load_gather(tv0, [jnp.minimum(lanes, n_vals0 - 1)])
        plsc.store_scatter(tb0, [pats], vals0)
        vals1 = plsc.load_gather(tv1, [lanes])
        plsc.store_scatter(tb1, [pats], vals1)

        outs = []
        for k in range(_CHUNKS):
            ins[k].wait()
            _lookup_rows(bufs[k], tb0, 0, n_cat0)
            _lookup_rows(bufs[k], tb1, n_cat0, n_cat0 + n_cat1)
            outs.append(pltpu.async_copy(
                bufs[k],
                out_hbm.at[:, pl.ds(base + k * _CHUNK_COLS, _CHUNK_COLS)],
                souts[k]))
        for o in outs:
            o.wait()

    return body(xt, tbl0, tbl1)


def kernel(x, cat_values_0, cat_values_1, cat_idx_0, cat_idx_1):
    out_t = _run(x.T, cat_values_0, cat_values_1,
                 cat_idx_0.shape[0], cat_idx_1.shape[0],
                 cat_values_0.shape[0])
    return out_t.T


# speedup vs baseline: 1.0427x; 1.0427x over previous
import functools

import jax
import jax.numpy as jnp
from jax import lax
from jax.experimental import pallas as pl
from jax.experimental.pallas import tpu as pltpu
from jax.experimental.pallas import tpu_sc as plsc

_LANES = 16
_N_WORKERS = 32
_CHUNK_COLS = 128
_CHUNKS = 4
_PAT_TABLE = 1048
_SHIFT = 20


def _lookup_rows(buf, tb, r_lo, r_hi):
    @plsc.parallel_loop(r_lo, r_hi, unroll=2)
    def body(r):
        for s in range(_CHUNK_COLS // _LANES):
            sl = pl.ds(s * _LANES, _LANES)
            idx = lax.shift_right_logical(
                plsc.bitcast(buf[r, sl], jnp.int32), _SHIFT)
            buf[r, sl] = plsc.load_gather(tb, [idx])


def _run(xt, tbl0, tbl1, n_cat0, n_cat1, n_vals0):
    n_params, batch = xt.shape
    stripe = batch // _N_WORKERS
    mesh = plsc.VectorSubcoreMesh(core_axis_name="c", subcore_axis_name="s")

    @functools.partial(
        pl.kernel,
        mesh=mesh,
        out_type=jax.ShapeDtypeStruct(xt.shape, jnp.float32),
        scratch_types=[
            pltpu.VMEM((n_params, _CHUNK_COLS), jnp.float32),
            pltpu.VMEM((n_params, _CHUNK_COLS), jnp.float32),
            pltpu.VMEM((n_params, _CHUNK_COLS), jnp.float32),
            pltpu.VMEM((n_params, _CHUNK_COLS), jnp.float32),
            pltpu.VMEM((8,), jnp.float32),
            pltpu.VMEM((_LANES,), jnp.float32),
            pltpu.VMEM((_PAT_TABLE,), jnp.float32),
            pltpu.VMEM((_PAT_TABLE,), jnp.float32),
            pltpu.SemaphoreType.DMA,
            pltpu.SemaphoreType.DMA,
            pltpu.SemaphoreType.DMA,
            pltpu.SemaphoreType.DMA,
            pltpu.SemaphoreType.DMA,
            pltpu.SemaphoreType.DMA,
            pltpu.SemaphoreType.DMA,
            pltpu.SemaphoreType.DMA,
        ],
        compiler_params=pltpu.CompilerParams(needs_layout_passes=False),
    )
    def body(xt_hbm, t0_hbm, t1_hbm, out_hbm, b0, b1, b2, b3, tv0, tv1,
             tb0, tb1, si0, si1, si2, si3, so0, so1, so2, so3):
        wid = lax.axis_index("s") * 2 + lax.axis_index("c")
        base = wid * stripe
        bufs = (b0, b1, b2, b3)
        sins = (si0, si1, si2, si3)
        souts = (so0, so1, so2, so3)

        pltpu.sync_copy(t0_hbm, tv0)
        pltpu.sync_copy(t1_hbm, tv1)

        ins = [
            pltpu.async_copy(
                xt_hbm.at[:, pl.ds(base + k * _CHUNK_COLS, _CHUNK_COLS)],
                bufs[k], sins[k])
            for k in range(_CHUNKS)
        ]

        lanes = lax.iota(jnp.int32, _LANES)
        pats = lax.shift_right_logical(
            plsc.bitcast(lanes.astype(jnp.float32), jnp.int32), _SHIFT)
        vals0 = plsc.load_gather(tv0, [jnp.minimum(lanes, n_vals0 - 1)])
        plsc.store_scatter(tb0, [pats], vals0)
        vals1 = plsc.load_gather(tv1, [lanes])
        plsc.store_scatter(tb1, [pats], vals1)

        outs = []
        for k in range(_CHUNKS):
            ins[k].wait()
            _lookup_rows(bufs[k], tb0, 0, n_cat0)
            _lookup_rows(bufs[k], tb1, n_cat0, n_cat0 + n_cat1)
            outs.append(pltpu.async_copy(
                bufs[k],
                out_hbm.at[:, pl.ds(base + k * _CHUNK_COLS, _CHUNK_COLS)],
                souts[k]))
        for o in outs:
            o.wait()

    return body(xt, tbl0, tbl1)


def kernel(x, cat_values_0, cat_values_1, cat_idx_0, cat_idx_1):
    out_t = _run(x.T, cat_values_0, cat_values_1,
                 cat_idx_0.shape[0], cat_idx_1.shape[0],
                 cat_values_0.shape[0])
    return out_t.T


# speedup vs baseline: 1.1147x; 1.0691x over previous
import functools

import jax
import jax.numpy as jnp
from jax import lax
from jax.experimental import pallas as pl
from jax.experimental.pallas import tpu as pltpu
from jax.experimental.pallas import tpu_sc as plsc

_LANES = 16
_N_WORKERS = 32
_CHUNK_COLS = 128
_CHUNKS = 4
_PAT_TABLE = 1048
_SHIFT = 20


def _lookup_rows(buf, tb, r_lo, r_hi):
    @plsc.parallel_loop(r_lo, r_hi, unroll=2)
    def body(r):
        for s in range(_CHUNK_COLS // _LANES):
            sl = pl.ds(s * _LANES, _LANES)
            idx = lax.shift_right_logical(
                plsc.bitcast(buf[r, sl], jnp.int32), _SHIFT)
            buf[r, sl] = plsc.load_gather(tb, [idx])


def _run(xt, tbl0, tbl1, n_cat0, n_cat1, n_vals0):
    n_params, batch = xt.shape
    stripe = batch // _N_WORKERS
    mesh = plsc.VectorSubcoreMesh(core_axis_name="c", subcore_axis_name="s")

    @functools.partial(
        pl.kernel,
        mesh=mesh,
        out_type=jax.ShapeDtypeStruct(xt.shape, jnp.float32),
        scratch_types=[
            pltpu.VMEM((n_params, _CHUNK_COLS), jnp.float32),
            pltpu.VMEM((n_params, _CHUNK_COLS), jnp.float32),
            pltpu.VMEM((n_params, _CHUNK_COLS), jnp.float32),
            pltpu.VMEM((n_params, _CHUNK_COLS), jnp.float32),
            pltpu.VMEM((8,), jnp.float32),
            pltpu.VMEM((_LANES,), jnp.float32),
            pltpu.VMEM((_PAT_TABLE,), jnp.float32),
            pltpu.VMEM((_PAT_TABLE,), jnp.float32),
            pltpu.SemaphoreType.DMA,
            pltpu.SemaphoreType.DMA,
            pltpu.SemaphoreType.DMA,
            pltpu.SemaphoreType.DMA,
            pltpu.SemaphoreType.DMA,
            pltpu.SemaphoreType.DMA,
            pltpu.SemaphoreType.DMA,
            pltpu.SemaphoreType.DMA,
            pltpu.SemaphoreType.DMA,
            pltpu.SemaphoreType.DMA,
        ],
        compiler_params=pltpu.CompilerParams(
            needs_layout_passes=False,
            skip_device_barrier=True,
            disable_semaphore_checks=True,
        ),
    )
    def body(xt_hbm, t0_hbm, t1_hbm, out_hbm, b0, b1, b2, b3, tv0, tv1,
             tb0, tb1, si0, si1, si2, si3, so0, so1, so2, so3, st0, st1):
        wid = lax.axis_index("s") * 2 + lax.axis_index("c")
        base = wid * stripe
        bufs = (b0, b1, b2, b3)
        sins = (si0, si1, si2, si3)
        souts = (so0, so1, so2, so3)

        tin0 = pltpu.async_copy(t0_hbm, tv0, st0)
        tin1 = pltpu.async_copy(t1_hbm, tv1, st1)

        ins = [
            pltpu.async_copy(
                xt_hbm.at[:, pl.ds(base + k * _CHUNK_COLS, _CHUNK_COLS)],
                bufs[k], sins[k])
            for k in range(_CHUNKS)
        ]

        tin0.wait()
        tin1.wait()
        lanes = lax.iota(jnp.int32, _LANES)
        pats = lax.shift_right_logical(
            plsc.bitcast(lanes.astype(jnp.float32), jnp.int32), _SHIFT)
        vals0 = plsc.load_gather(tv0, [jnp.minimum(lanes, n_vals0 - 1)])
        plsc.store_scatter(tb0, [pats], vals0)
        vals1 = plsc.load_gather(tv1, [lanes])
        plsc.store_scatter(tb1, [pats], vals1)

        outs = []
        for k in range(_CHUNKS):
            ins[k].wait()
            _lookup_rows(bufs[k], tb0, 0, n_cat0)
            _lookup_rows(bufs[k], tb1, n_cat0, n_cat0 + n_cat1)
            outs.append(pltpu.async_copy(
                bufs[k],
                out_hbm.at[:, pl.ds(base + k * _CHUNK_COLS, _CHUNK_COLS)],
                souts[k]))
        for o in outs:
            o.wait()

    return body(xt, tbl0, tbl1)


def kernel(x, cat_values_0, cat_values_1, cat_idx_0, cat_idx_1):
    out_t = _run(x.T, cat_values_0, cat_values_1,
                 cat_idx_0.shape[0], cat_idx_1.shape[0],
                 cat_values_0.shape[0])
    return out_t.T


# speedup vs baseline: 1.1662x; 1.0462x over previous
import functools

import jax
import jax.numpy as jnp
from jax import lax
from jax.experimental import pallas as pl
from jax.experimental.pallas import tpu as pltpu
from jax.experimental.pallas import tpu_sc as plsc

_LANES = 16
_N_WORKERS = 32
_CHUNK_COLS = 128
_CHUNKS = 4
_PAT_TABLE = 1048
_SHIFT = 20


def _lookup_rows(buf, tb, n_cat0, n_cat):
    @plsc.parallel_loop(0, n_cat, unroll=1)
    def body(r):
        off = jnp.where(r < n_cat0, 0, _PAT_TABLE)
        tbr = tb.at[pl.ds(off, _PAT_TABLE)]
        for s in range(_CHUNK_COLS // _LANES):
            sl = pl.ds(s * _LANES, _LANES)
            idx = lax.shift_right_logical(
                plsc.bitcast(buf[r, sl], jnp.int32), _SHIFT)
            buf[r, sl] = plsc.load_gather(tbr, [idx])


def _run(xt, tbl0, tbl1, n_cat0, n_cat1, n_vals0):
    n_params, batch = xt.shape
    stripe = batch // _N_WORKERS
    mesh = plsc.VectorSubcoreMesh(core_axis_name="c", subcore_axis_name="s")

    @functools.partial(
        pl.kernel,
        mesh=mesh,
        out_type=jax.ShapeDtypeStruct(xt.shape, jnp.float32),
        scratch_types=[
            pltpu.VMEM((n_params, _CHUNK_COLS), jnp.float32),
            pltpu.VMEM((n_params, _CHUNK_COLS), jnp.float32),
            pltpu.VMEM((n_params, _CHUNK_COLS), jnp.float32),
            pltpu.VMEM((n_params, _CHUNK_COLS), jnp.float32),
            pltpu.VMEM((8,), jnp.float32),
            pltpu.VMEM((_LANES,), jnp.float32),
            pltpu.VMEM((2 * _PAT_TABLE,), jnp.float32),
            pltpu.SemaphoreType.DMA,
            pltpu.SemaphoreType.DMA,
            pltpu.SemaphoreType.DMA,
            pltpu.SemaphoreType.DMA,
            pltpu.SemaphoreType.DMA,
            pltpu.SemaphoreType.DMA,
            pltpu.SemaphoreType.DMA,
            pltpu.SemaphoreType.DMA,
            pltpu.SemaphoreType.DMA,
            pltpu.SemaphoreType.DMA,
        ],
        compiler_params=pltpu.CompilerParams(
            needs_layout_passes=False,
            skip_device_barrier=True,
            disable_semaphore_checks=True,
        ),
    )
    def body(xt_hbm, t0_hbm, t1_hbm, out_hbm, b0, b1, b2, b3, tv0, tv1,
             tb, si0, si1, si2, si3, so0, so1, so2, so3, st0, st1):
        wid = lax.axis_index("s") * 2 + lax.axis_index("c")
        base = wid * stripe
        bufs = (b0, b1, b2, b3)
        sins = (si0, si1, si2, si3)
        souts = (so0, so1, so2, so3)

        tin0 = pltpu.async_copy(t0_hbm, tv0, st0)
        tin1 = pltpu.async_copy(t1_hbm, tv1, st1)

        ins = [
            pltpu.async_copy(
                xt_hbm.at[:, pl.ds(base + k * _CHUNK_COLS, _CHUNK_COLS)],
                bufs[k], sins[k])
            for k in range(_CHUNKS)
        ]

        tin0.wait()
        tin1.wait()
        lanes = lax.iota(jnp.int32, _LANES)
        pats = lax.shift_right_logical(
            plsc.bitcast(lanes.astype(jnp.float32), jnp.int32), _SHIFT)
        vals0 = plsc.load_gather(tv0, [jnp.minimum(lanes, n_vals0 - 1)])
        plsc.store_scatter(tb, [pats], vals0)
        vals1 = plsc.load_gather(tv1, [lanes])
        plsc.store_scatter(tb, [pats + _PAT_TABLE], vals1)

        outs = []
        for k in range(_CHUNKS):
            ins[k].wait()
            _lookup_rows(bufs[k], tb, n_cat0, n_cat0 + n_cat1)
            outs.append(pltpu.async_copy(
                bufs[k],
                out_hbm.at[:, pl.ds(base + k * _CHUNK_COLS, _CHUNK_COLS)],
                souts[k]))
        for o in outs:
            o.wait()

    return body(xt, tbl0, tbl1)


def kernel(x, cat_values_0, cat_values_1, cat_idx_0, cat_idx_1):
    out_t = _run(x.T, cat_values_0, cat_values_1,
                 cat_idx_0.shape[0], cat_idx_1.shape[0],
                 cat_values_0.shape[0])
    return out_t.T


# speedup vs baseline: 1.1701x; 1.0033x over previous
import functools

import jax
import jax.numpy as jnp
from jax import lax
from jax.experimental import pallas as pl
from jax.experimental.pallas import tpu as pltpu
from jax.experimental.pallas import tpu_sc as plsc

_LANES = 16
_N_WORKERS = 32
_CHUNK_COLS = 256
_CHUNKS = 2
_PAT_TABLE = 1048
_SHIFT = 20


def _lookup_rows(buf, tb, n_cat0, n_cat):
    @plsc.parallel_loop(0, n_cat, unroll=1)
    def body(r):
        off = jnp.where(r < n_cat0, 0, _PAT_TABLE)
        tbr = tb.at[pl.ds(off, _PAT_TABLE)]
        for s in range(_CHUNK_COLS // _LANES):
            sl = pl.ds(s * _LANES, _LANES)
            idx = lax.shift_right_logical(
                plsc.bitcast(buf[r, sl], jnp.int32), _SHIFT)
            buf[r, sl] = plsc.load_gather(tbr, [idx])


def _run(xt, tbl0, tbl1, n_cat0, n_cat1, n_vals0):
    n_params, batch = xt.shape
    stripe = batch // _N_WORKERS
    mesh = plsc.VectorSubcoreMesh(core_axis_name="c", subcore_axis_name="s")

    @functools.partial(
        pl.kernel,
        mesh=mesh,
        out_type=jax.ShapeDtypeStruct(xt.shape, jnp.float32),
        scratch_types=[
            pltpu.VMEM((n_params, _CHUNK_COLS), jnp.float32),
            pltpu.VMEM((n_params, _CHUNK_COLS), jnp.float32),
            pltpu.VMEM((8,), jnp.float32),
            pltpu.VMEM((_LANES,), jnp.float32),
            pltpu.VMEM((2 * _PAT_TABLE,), jnp.float32),
            pltpu.SemaphoreType.DMA,
            pltpu.SemaphoreType.DMA,
            pltpu.SemaphoreType.DMA,
            pltpu.SemaphoreType.DMA,
            pltpu.SemaphoreType.DMA,
            pltpu.SemaphoreType.DMA,
        ],
        compiler_params=pltpu.CompilerParams(
            needs_layout_passes=False,
            skip_device_barrier=True,
            disable_semaphore_checks=True,
        ),
    )
    def body(xt_hbm, t0_hbm, t1_hbm, out_hbm, b0, b1, tv0, tv1,
             tb, si0, si1, so0, so1, st0, st1):
        wid = lax.axis_index("s") * 2 + lax.axis_index("c")
        base = wid * stripe
        bufs = (b0, b1)
        sins = (si0, si1)
        souts = (so0, so1)

        tin0 = pltpu.async_copy(t0_hbm, tv0, st0)
        tin1 = pltpu.async_copy(t1_hbm, tv1, st1)

        ins = [
            pltpu.async_copy(
                xt_hbm.at[:, pl.ds(base + k * _CHUNK_COLS, _CHUNK_COLS)],
                bufs[k], sins[k])
            for k in range(_CHUNKS)
        ]

        tin0.wait()
        tin1.wait()
        lanes = lax.iota(jnp.int32, _LANES)
        pats = lax.shift_right_logical(
            plsc.bitcast(lanes.astype(jnp.float32), jnp.int32), _SHIFT)
        vals0 = plsc.load_gather(tv0, [jnp.minimum(lanes, n_vals0 - 1)])
        plsc.store_scatter(tb, [pats], vals0)
        vals1 = plsc.load_gather(tv1, [lanes])
        plsc.store_scatter(tb, [pats + _PAT_TABLE], vals1)

        outs = []
        for k in range(_CHUNKS):
            ins[k].wait()
            _lookup_rows(bufs[k], tb, n_cat0, n_cat0 + n_cat1)
            outs.append(pltpu.async_copy(
                bufs[k],
                out_hbm.at[:, pl.ds(base + k * _CHUNK_COLS, _CHUNK_COLS)],
                souts[k]))
        for o in outs:
            o.wait()

    return body(xt, tbl0, tbl1)


def kernel(x, cat_values_0, cat_values_1, cat_idx_0, cat_idx_1):
    out_t = _run(x.T, cat_values_0, cat_values_1,
                 cat_idx_0.shape[0], cat_idx_1.shape[0],
                 cat_values_0.shape[0])
    return out_t.T


# speedup vs baseline: 1.1702x; 1.0001x over previous
import functools

import jax
import jax.numpy as jnp
from jax import lax
from jax.experimental import pallas as pl
from jax.experimental.pallas import tpu as pltpu
from jax.experimental.pallas import tpu_sc as plsc

_LANES = 16
_N_WORKERS = 32
_CHUNK_COLS = 256
_CHUNKS = 2
_PAT_TABLE = 1048
_SHIFT = 20


def _lookup_rows(buf, tb, n_cat0, n_cat):
    @plsc.parallel_loop(0, n_cat, unroll=1)
    def body(r):
        off = jnp.where(r < n_cat0, 0, _PAT_TABLE)
        tbr = tb.at[pl.ds(off, _PAT_TABLE)]
        for s in range(_CHUNK_COLS // _LANES):
            sl = pl.ds(s * _LANES, _LANES)
            idx = lax.shift_right_logical(
                plsc.bitcast(buf[r, sl], jnp.int32), _SHIFT)
            buf[r, sl] = plsc.load_gather(tbr, [idx])


def _run(xt, tbl0, tbl1, n_cat0, n_cat1, n_vals0):
    n_params, batch = xt.shape
    stripe = batch // _N_WORKERS
    mesh = plsc.VectorSubcoreMesh(core_axis_name="c", subcore_axis_name="s")

    @functools.partial(
        pl.kernel,
        mesh=mesh,
        out_type=jax.ShapeDtypeStruct(xt.shape, jnp.float32),
        scratch_types=[
            pltpu.VMEM((n_params, _CHUNK_COLS), jnp.float32),
            pltpu.VMEM((n_params, _CHUNK_COLS), jnp.float32),
            pltpu.VMEM((8,), jnp.float32),
            pltpu.VMEM((_LANES,), jnp.float32),
            pltpu.VMEM((2 * _PAT_TABLE,), jnp.float32),
            pltpu.SemaphoreType.DMA,
            pltpu.SemaphoreType.DMA,
            pltpu.SemaphoreType.DMA,
            pltpu.SemaphoreType.DMA,
            pltpu.SemaphoreType.DMA,
            pltpu.SemaphoreType.DMA,
        ],
        compiler_params=pltpu.CompilerParams(
            needs_layout_passes=False,
            skip_device_barrier=True,
            disable_semaphore_checks=True,
            disable_bounds_checks=True,
        ),
    )
    def body(xt_hbm, t0_hbm, t1_hbm, out_hbm, b0, b1, tv0, tv1,
             tb, si0, si1, so0, so1, st0, st1):
        wid = lax.axis_index("s") * 2 + lax.axis_index("c")
        base = wid * stripe
        bufs = (b0, b1)
        sins = (si0, si1)
        souts = (so0, so1)

        tin0 = pltpu.async_copy(t0_hbm, tv0, st0)
        tin1 = pltpu.async_copy(t1_hbm, tv1, st1)

        ins = [
            pltpu.async_copy(
                xt_hbm.at[:, pl.ds(base + k * _CHUNK_COLS, _CHUNK_COLS)],
                bufs[k], sins[k])
            for k in range(_CHUNKS)
        ]

        tin0.wait()
        tin1.wait()
        lanes = lax.iota(jnp.int32, _LANES)
        pats = lax.shift_right_logical(
            plsc.bitcast(lanes.astype(jnp.float32), jnp.int32), _SHIFT)
        vals0 = plsc.load_gather(tv0, [jnp.minimum(lanes, n_vals0 - 1)])
        plsc.store_scatter(tb, [pats], vals0)
        vals1 = plsc.load_gather(tv1, [lanes])
        plsc.store_scatter(tb, [pats + _PAT_TABLE], vals1)

        outs = []
        for k in range(_CHUNKS):
            ins[k].wait()
            _lookup_rows(bufs[k], tb, n_cat0, n_cat0 + n_cat1)
            outs.append(pltpu.async_copy(
                bufs[k],
                out_hbm.at[:, pl.ds(base + k * _CHUNK_COLS, _CHUNK_COLS)],
                souts[k]))
        for o in outs:
            o.wait()

    return body(xt, tbl0, tbl1)


def kernel(x, cat_values_0, cat_values_1, cat_idx_0, cat_idx_1):
    out_t = _run(x.T, cat_values_0, cat_values_1,
                 cat_idx_0.shape[0], cat_idx_1.shape[0],
                 cat_values_0.shape[0])
    return out_t.T
